# baseline (speedup 1.0000x reference)
import functools

import jax
import jax.numpy as jnp
from jax import lax
from jax.experimental import pallas as pl
from jax.experimental.pallas import tpu as pltpu
from jax.experimental.pallas import tpu_sc as plsc

_N = 10000
_E = 160000
_D = 256
_DH = 128
_ALPHA = 0.5

_NS = 16
_CH = 80
_PER_TILE = _E // _NS
_N_CHUNKS = _PER_TILE // _CH
_NPAD = 10240
_ROWS_PER_TILE = _NPAD // _NS


def _sc_aggregate(x0, x1, edge_index_flat, zeros):
    mesh = plsc.VectorSubcoreMesh(core_axis_name="c", subcore_axis_name="s")

    @functools.partial(
        pl.kernel,
        out_type=jax.ShapeDtypeStruct((2, 2, _NPAD, _DH), jnp.float32),
        mesh=mesh,
        scratch_types=[
            [pltpu.VMEM((_CH,), jnp.int32) for _ in range(2)],
            [pltpu.VMEM((_CH,), jnp.int32) for _ in range(4)],
            [pltpu.VMEM((_CH, _DH), jnp.float32) for _ in range(4)],
            pltpu.VMEM_SHARED((_NPAD, _DH), jnp.float32),
            [pltpu.SemaphoreType.DMA for _ in range(2)],
            [pltpu.SemaphoreType.DMA for _ in range(4)],
            [pltpu.SemaphoreType.DMA for _ in range(4)],
        ],
    )
    def agg_kernel(x0_hbm, x1_hbm, ei_hbm, z_hbm, out_hbm,
                   gidx_v, sidx_v, rows, acc, gsem, isem, ssem):
        c = lax.axis_index("c")
        s = lax.axis_index("s")
        row0 = s * _ROWS_PER_TILE
        tbase = s * _PER_TILE

        def fetch_gidx(i, b):
            pltpu.sync_copy(ei_hbm.at[pl.ds(c * _E + tbase + i * _CH, _CH)],
                            gidx_v[b])

        def sidx_copy(i, b):
            return pltpu.make_async_copy(
                ei_hbm.at[pl.ds((1 - c) * _E + tbase + i * _CH, _CH)],
                sidx_v[b], isem[b])

        def scat_desc(b):
            return pltpu.make_async_copy(rows[b], acc.at[sidx_v[b]], ssem[b])

        pltpu.sync_copy(z_hbm, acc.at[pl.ds(row0, _ROWS_PER_TILE)])

        for p, x_hbm in ((0, x0_hbm), (1, x1_hbm)):
            plsc.subcore_barrier()

            fetch_gidx(0, 0)
            pltpu.async_copy(x_hbm.at[gidx_v[0]], rows[0], gsem[0])
            sidx_copy(0, 0).start()

            @pl.loop(0, _N_CHUNKS // 4)
            def _(g):
                for b in range(4):
                    i = g * 4 + b
                    nxt = i + 1
                    nb = (b + 1) % 4

                    @pl.when(nxt < _N_CHUNKS)
                    def _():
                        @pl.when(i >= 3)
                        def _():
                            scat_desc(nb).wait()
                        gp = (b + 1) % 2
                        fetch_gidx(nxt, gp)
                        pltpu.async_copy(x_hbm.at[gidx_v[gp]], rows[nb],
                                         gsem[gp])
                        sidx_copy(nxt, nb).start()

                    pltpu.make_async_copy(
                        x_hbm.at[gidx_v[b % 2]], rows[b], gsem[b % 2]).wait()
                    sidx_copy(i, b).wait()
                    pltpu.async_copy(rows[b], acc.at[sidx_v[b]], ssem[b],
                                     add=True)

            last = _N_CHUNKS - 1
            pltpu.make_async_copy(
                x_hbm.at[gidx_v[0]], rows[0], gsem[0]).wait()
            sidx_copy(last, 0).wait()
            pltpu.async_copy(rows[0], acc.at[sidx_v[0]], ssem[0], add=True)
            for b in (1, 2, 3, 0):
                scat_desc(b).wait()

            plsc.subcore_barrier()
            pltpu.sync_copy(acc.at[pl.ds(row0, _ROWS_PER_TILE)],
                            out_hbm.at[c, p, pl.ds(row0, _ROWS_PER_TILE)])
            if p == 0:
                pltpu.sync_copy(z_hbm, acc.at[pl.ds(row0, _ROWS_PER_TILE)])

    return agg_kernel(x0, x1, edge_index_flat, zeros)


_BLK = 1000


def _mlp_body(x_ref, as0_ref, as1_ref, ad0_ref, ad1_ref,
              w1s, b1s, w2s, b2s, w1d, b1d, w2d, b2d, o_ref):
    xs = x_ref[...]
    bf = jnp.bfloat16
    hs = (xs + jnp.concatenate([as0_ref[...], as1_ref[...]], axis=-1)).astype(bf)
    hd = (xs + jnp.concatenate([ad0_ref[...], ad1_ref[...]], axis=-1)).astype(bf)
    ts = jnp.maximum(
        jnp.dot(hs, w1s[...].astype(bf),
                preferred_element_type=jnp.float32) + b1s[...], 0.0).astype(bf)
    ys = jnp.dot(ts, w2s[...].astype(bf),
                 preferred_element_type=jnp.float32) + b2s[...]
    td = jnp.maximum(
        jnp.dot(hd, w1d[...].astype(bf),
                preferred_element_type=jnp.float32) + b1d[...], 0.0).astype(bf)
    yd = jnp.dot(td, w2d[...].astype(bf),
                 preferred_element_type=jnp.float32) + b2d[...]
    o_ref[...] = (1.0 - _ALPHA) * ys + _ALPHA * yd


def _tc_mlp(x, aggs0, aggs1, aggd0, aggd1,
            W1s, b1s, W2s, b2s, W1d, b1d, W2d, b2d):
    half_spec = pl.BlockSpec((_BLK, _DH), lambda i: (i, 0))
    w_spec = pl.BlockSpec((_D, _D), lambda i: (0, 0))
    b_spec = pl.BlockSpec((1, _D), lambda i: (0, 0))
    return pl.pallas_call(
        _mlp_body,
        grid=(_N // _BLK,),
        in_specs=[
            pl.BlockSpec((_BLK, _D), lambda i: (i, 0)),
            half_spec, half_spec, half_spec, half_spec,
            w_spec, b_spec, w_spec, b_spec,
            w_spec, b_spec, w_spec, b_spec,
        ],
        out_specs=pl.BlockSpec((_BLK, _D), lambda i: (i, 0)),
        out_shape=jax.ShapeDtypeStruct((_N, _D), jnp.float32),
    )(x, aggs0, aggs1, aggd0, aggd1,
      W1s, b1s.reshape(1, _D), W2s, b2s.reshape(1, _D),
      W1d, b1d.reshape(1, _D), W2d, b2d.reshape(1, _D))


def kernel(x, edge_index, W1s, b1s, W2s, b2s, W1d, b1d, W2d, b2d):
    x0 = x[:, :_DH]
    x1 = x[:, _DH:]
    zeros = jnp.zeros((_ROWS_PER_TILE, _DH), jnp.float32)
    agg = _sc_aggregate(x0, x1, edge_index.reshape(-1), zeros)
    return _tc_mlp(x, agg[0, 0, :_N], agg[0, 1, :_N], agg[1, 0, :_N],
                   agg[1, 1, :_N],
                   W1s, b1s, W2s, b2s, W1d, b1d, W2d, b2d)


# speedup vs baseline: 1.1704x; 1.1704x over previous
import functools

import jax
import jax.numpy as jnp
from jax import lax
from jax.experimental import pallas as pl
from jax.experimental.pallas import tpu as pltpu
from jax.experimental.pallas import tpu_sc as plsc

_N = 10000
_E = 160000
_D = 256
_DH = 128
_ALPHA = 0.5

_NS = 16
_CH = 80
_PER_TILE = _E // _NS
_N_CHUNKS = _PER_TILE // _CH
_NPAD = 10240
_ROWS_PER_TILE = _NPAD // _NS


def _sc_aggregate(x0, x1, edge_index_flat, zeros):
    mesh = plsc.VectorSubcoreMesh(core_axis_name="c", subcore_axis_name="s")

    @functools.partial(
        pl.kernel,
        out_type=jax.ShapeDtypeStruct((2, 2, _NPAD, _DH), jnp.float32),
        mesh=mesh,
        scratch_types=[
            [pltpu.VMEM((_CH,), jnp.int32) for _ in range(4)],
            [pltpu.VMEM((_CH,), jnp.int32) for _ in range(4)],
            [pltpu.VMEM((_CH, _DH), jnp.float32) for _ in range(4)],
            pltpu.VMEM_SHARED((_NPAD, _DH), jnp.float32),
            [pltpu.SemaphoreType.DMA for _ in range(2)],
            [pltpu.SemaphoreType.DMA for _ in range(4)],
            [pltpu.SemaphoreType.DMA for _ in range(4)],
            [pltpu.SemaphoreType.DMA for _ in range(4)],
        ],
    )
    def agg_kernel(x0_hbm, x1_hbm, ei_hbm, z_hbm, out_hbm,
                   gidx_v, sidx_v, rows, acc, gsem, isem, ssem, gisem):
        c = lax.axis_index("c")
        s = lax.axis_index("s")
        row0 = s * _ROWS_PER_TILE
        tbase = s * _PER_TILE

        def gidx_copy(i, b):
            return pltpu.make_async_copy(
                ei_hbm.at[pl.ds(c * _E + tbase + i * _CH, _CH)],
                gidx_v[b], gisem[b])

        def sidx_copy(i, b):
            return pltpu.make_async_copy(
                ei_hbm.at[pl.ds((1 - c) * _E + tbase + i * _CH, _CH)],
                sidx_v[b], isem[b])

        def scat_desc(b):
            return pltpu.make_async_copy(rows[b], acc.at[sidx_v[b]], ssem[b])

        pltpu.sync_copy(z_hbm, acc.at[pl.ds(row0, _ROWS_PER_TILE)])

        for p, x_hbm in ((0, x0_hbm), (1, x1_hbm)):
            plsc.subcore_barrier()

            gidx_copy(0, 0).start()
            gidx_copy(0, 0).wait()
            pltpu.async_copy(x_hbm.at[gidx_v[0]], rows[0], gsem[0])
            gidx_copy(1, 1).start()
            sidx_copy(0, 0).start()

            @pl.loop(0, _N_CHUNKS // 4)
            def _(g):
                for b in range(4):
                    i = g * 4 + b
                    nxt = i + 1
                    nb = (b + 1) % 4
                    nb2 = (b + 2) % 4

                    @pl.when(i + 2 < _N_CHUNKS)
                    def _():
                        gidx_copy(i + 2, nb2).start()

                    @pl.when(nxt < _N_CHUNKS)
                    def _():
                        @pl.when(i >= 3)
                        def _():
                            scat_desc(nb).wait()
                        gidx_copy(nxt, nb).wait()
                        pltpu.async_copy(x_hbm.at[gidx_v[nb]], rows[nb],
                                         gsem[(b + 1) % 2])
                        sidx_copy(nxt, nb).start()

                    pltpu.make_async_copy(
                        x_hbm.at[gidx_v[b]], rows[b], gsem[b % 2]).wait()
                    sidx_copy(i, b).wait()
                    pltpu.async_copy(rows[b], acc.at[sidx_v[b]], ssem[b],
                                     add=True)

            last = _N_CHUNKS - 1
            pltpu.make_async_copy(
                x_hbm.at[gidx_v[0]], rows[0], gsem[0]).wait()
            sidx_copy(last, 0).wait()
            pltpu.async_copy(rows[0], acc.at[sidx_v[0]], ssem[0], add=True)
            for b in (1, 2, 3, 0):
                scat_desc(b).wait()

            plsc.subcore_barrier()
            pltpu.sync_copy(acc.at[pl.ds(row0, _ROWS_PER_TILE)],
                            out_hbm.at[c, p, pl.ds(row0, _ROWS_PER_TILE)])
            if p == 0:
                pltpu.sync_copy(z_hbm, acc.at[pl.ds(row0, _ROWS_PER_TILE)])

    return agg_kernel(x0, x1, edge_index_flat, zeros)


_BLK = 1000


def _mlp_body(x_ref, as0_ref, as1_ref, ad0_ref, ad1_ref,
              w1s, b1s, w2s, b2s, w1d, b1d, w2d, b2d, o_ref):
    xs = x_ref[...]
    hs = xs + jnp.concatenate([as0_ref[...], as1_ref[...]], axis=-1)
    hd = xs + jnp.concatenate([ad0_ref[...], ad1_ref[...]], axis=-1)
    ts = jnp.maximum(
        jnp.dot(hs, w1s[...], preferred_element_type=jnp.float32) + b1s[...], 0.0)
    ys = jnp.dot(ts, w2s[...], preferred_element_type=jnp.float32) + b2s[...]
    td = jnp.maximum(
        jnp.dot(hd, w1d[...], preferred_element_type=jnp.float32) + b1d[...], 0.0)
    yd = jnp.dot(td, w2d[...], preferred_element_type=jnp.float32) + b2d[...]
    o_ref[...] = (1.0 - _ALPHA) * ys + _ALPHA * yd


def _tc_mlp(x, aggs0, aggs1, aggd0, aggd1,
            W1s, b1s, W2s, b2s, W1d, b1d, W2d, b2d):
    half_spec = pl.BlockSpec((_BLK, _DH), lambda i: (i, 0))
    w_spec = pl.BlockSpec((_D, _D), lambda i: (0, 0))
    b_spec = pl.BlockSpec((1, _D), lambda i: (0, 0))
    return pl.pallas_call(
        _mlp_body,
        grid=(_N // _BLK,),
        in_specs=[
            pl.BlockSpec((_BLK, _D), lambda i: (i, 0)),
            half_spec, half_spec, half_spec, half_spec,
            w_spec, b_spec, w_spec, b_spec,
            w_spec, b_spec, w_spec, b_spec,
        ],
        out_specs=pl.BlockSpec((_BLK, _D), lambda i: (i, 0)),
        out_shape=jax.ShapeDtypeStruct((_N, _D), jnp.float32),
    )(x, aggs0, aggs1, aggd0, aggd1,
      W1s, b1s.reshape(1, _D), W2s, b2s.reshape(1, _D),
      W1d, b1d.reshape(1, _D), W2d, b2d.reshape(1, _D))


def kernel(x, edge_index, W1s, b1s, W2s, b2s, W1d, b1d, W2d, b2d):
    x0 = x[:, :_DH]
    x1 = x[:, _DH:]
    zeros = jnp.zeros((_ROWS_PER_TILE, _DH), jnp.float32)
    agg = _sc_aggregate(x0, x1, edge_index.reshape(-1), zeros)
    return _tc_mlp(x, agg[0, 0, :_N], agg[0, 1, :_N], agg[1, 0, :_N],
                   agg[1, 1, :_N],
                   W1s, b1s, W2s, b2s, W1d, b1d, W2d, b2d)
